# baseline (speedup 1.0000x reference)
import functools

import jax
import jax.numpy as jnp
from jax import lax
from jax.experimental import pallas as pl
from jax.experimental.pallas import tpu as pltpu
from jax.experimental.pallas import tpu_sc as plsc

H = 24
W = 24
HALF = 64
EMBED = 128
P = H * W
R = 2 * H - 1
NW = 32
PW = P // NW


JB = 12


def _build_body(tx_ref, ty_ref, out_ref):
    j0 = pl.program_id(0) * JB
    cols = lax.broadcasted_iota(jnp.int32, (JB, W, R), 2)
    aa = lax.broadcasted_iota(jnp.int32, (JB, W, R), 0)
    ll = lax.broadcasted_iota(jnp.int32, (JB, W, R), 1)
    onehot = jnp.where(cols == 23 - j0 - aa + ll, 1.0, 0.0).astype(jnp.float32)
    shifted = jax.lax.dot_general(
        onehot.reshape(JB * W, R),
        ty_ref[...],
        (((1,), (0,)), ((), ())),
        precision=jax.lax.Precision.HIGHEST,
        preferred_element_type=jnp.float32,
    ).reshape(JB, W, HALF)
    out_ref[:, :, :, :HALF] = jnp.broadcast_to(
        tx_ref[...][None, :, None, :], (JB, R, W, HALF)
    )
    out_ref[:, :, :, HALF:] = jnp.broadcast_to(
        shifted[:, None, :, :], (JB, R, W, HALF)
    )


G = 6
NG = H // G
C = 2
KC = H // C
NUNITS = W * NG * C
UPW = NUNITS // NW
WROWS = KC + G - 1


def _sc_body(s_hbm, out_hbm, buf0, buf1, lsem0, lsem1, wsem):
    c_id = lax.axis_index("c")
    s_id = lax.axis_index("s")
    wid = s_id * 2 + c_id
    bufs = (buf0, buf1)
    lsems = (lsem0, lsem1)

    def unit(m):
        u = wid * UPW + m
        jj = lax.div(u, NG * C)
        rem = lax.rem(u, NG * C)
        g = lax.div(rem, C)
        c = lax.rem(rem, C)
        return jj, g * G, c * KC

    def start_load(m):
        jj, i0, k0 = unit(m)
        return pltpu.async_copy(
            s_hbm.at[jj, pl.ds(H - G - i0 + k0, WROWS)],
            bufs[m % 2],
            lsems[m % 2],
        )

    loads = [start_load(0), start_load(1)]
    for m in range(UPW):
        b = m % 2
        loads[b].wait()
        jj, i0, k0 = unit(m)
        writes = []
        for d in range(G):
            p = (i0 + d) * W + jj
            writes.append(
                pltpu.async_copy(
                    bufs[b].at[pl.ds(G - 1 - d, KC)],
                    out_hbm.at[p, pl.ds(k0, KC)],
                    wsem,
                )
            )
        for wr in writes:
            wr.wait()
        if m + 2 < UPW:
            loads[b] = start_load(m + 2)


@functools.cache
def _sc_call():
    mesh = plsc.VectorSubcoreMesh(
        core_axis_name="c", subcore_axis_name="s", num_cores=2, num_subcores=16
    )
    return pl.kernel(
        _sc_body,
        out_type=jax.ShapeDtypeStruct((P, H, W, EMBED), jnp.float32),
        mesh=mesh,
        scratch_types=[
            pltpu.VMEM((WROWS, W, EMBED), jnp.float32),
            pltpu.VMEM((WROWS, W, EMBED), jnp.float32),
            pltpu.SemaphoreType.DMA,
            pltpu.SemaphoreType.DMA,
            pltpu.SemaphoreType.DMA,
        ],
    )


@jax.jit
def kernel(table_x, table_y):
    s_tab = pl.pallas_call(
        _build_body,
        grid=(W // JB,),
        in_specs=[
            pl.BlockSpec((R, HALF), lambda d: (0, 0)),
            pl.BlockSpec((R, HALF), lambda d: (0, 0)),
        ],
        out_specs=pl.BlockSpec((JB, R, W, EMBED), lambda d: (d, 0, 0, 0)),
        out_shape=jax.ShapeDtypeStruct((W, R, W, EMBED), jnp.float32),
        compiler_params=pltpu.CompilerParams(
            dimension_semantics=("parallel",)
        ),
    )(table_x, table_y)
    out128 = _sc_call()(s_tab)
    return out128.reshape(P, P, EMBED)


# speedup vs baseline: 1.0443x; 1.0443x over previous
import functools

import jax
import jax.numpy as jnp
from jax import lax
from jax.experimental import pallas as pl
from jax.experimental.pallas import tpu as pltpu
from jax.experimental.pallas import tpu_sc as plsc

H = 24
W = 24
HALF = 64
EMBED = 128
P = H * W
R = 2 * H - 1
NW = 32


JB = 12


def _build_body(tx_ref, ty_ref, out_ref):
    j0 = pl.program_id(0) * JB
    cols = lax.broadcasted_iota(jnp.int32, (JB, W, R), 2)
    aa = lax.broadcasted_iota(jnp.int32, (JB, W, R), 0)
    ll = lax.broadcasted_iota(jnp.int32, (JB, W, R), 1)
    onehot = jnp.where(cols == 23 - j0 - aa + ll, 1.0, 0.0).astype(jnp.float32)
    shifted = jax.lax.dot_general(
        onehot.reshape(JB * W, R),
        ty_ref[...],
        (((1,), (0,)), ((), ())),
        precision=jax.lax.Precision.HIGHEST,
        preferred_element_type=jnp.float32,
    ).reshape(JB, W, HALF)
    out_ref[:, :, :, :HALF] = jnp.broadcast_to(
        tx_ref[...][None, :, None, :], (JB, R, W, HALF)
    )
    out_ref[:, :, :, HALF:] = jnp.broadcast_to(
        shifted[:, None, :, :], (JB, R, W, HALF)
    )


G = 12
NG = H // G
C = 2
KC = H // C
NUNITS = W * NG * C
UPW = NUNITS // NW
WROWS = KC + G - 1


def _sc_body(s_hbm, out_hbm, buf_v, wsem):
    c_id = lax.axis_index("c")
    s_id = lax.axis_index("s")
    wid = s_id * 2 + c_id

    writes = []
    for m in range(UPW):
        u = wid * UPW + m
        jj = lax.div(u, NG * C)
        rem = lax.rem(u, NG * C)
        g = lax.div(rem, C)
        c = lax.rem(rem, C)
        i0 = g * G
        k0 = c * KC
        for wr in writes:
            wr.wait()
        writes = []
        pltpu.sync_copy(
            s_hbm.at[jj, pl.ds(KC - i0 + k0, WROWS)], buf_v
        )
        for d in range(G):
            p = (i0 + d) * W + jj
            writes.append(
                pltpu.async_copy(
                    buf_v.at[pl.ds(G - 1 - d, KC)],
                    out_hbm.at[p, pl.ds(k0, KC)],
                    wsem,
                )
            )
    for wr in writes:
        wr.wait()


@functools.cache
def _sc_call():
    mesh = plsc.VectorSubcoreMesh(
        core_axis_name="c", subcore_axis_name="s", num_cores=2, num_subcores=16
    )
    return pl.kernel(
        _sc_body,
        out_type=jax.ShapeDtypeStruct((P, H, W, EMBED), jnp.float32),
        mesh=mesh,
        scratch_types=[
            pltpu.VMEM((WROWS, W, EMBED), jnp.float32),
            pltpu.SemaphoreType.DMA,
        ],
    )


@jax.jit
def kernel(table_x, table_y):
    s_tab = pl.pallas_call(
        _build_body,
        grid=(W // JB,),
        in_specs=[
            pl.BlockSpec((R, HALF), lambda d: (0, 0)),
            pl.BlockSpec((R, HALF), lambda d: (0, 0)),
        ],
        out_specs=pl.BlockSpec((JB, R, W, EMBED), lambda d: (d, 0, 0, 0)),
        out_shape=jax.ShapeDtypeStruct((W, R, W, EMBED), jnp.float32),
        compiler_params=pltpu.CompilerParams(
            dimension_semantics=("parallel",)
        ),
    )(table_x, table_y)
    out128 = _sc_call()(s_tab)
    return out128.reshape(P, P, EMBED)
